# baseline (speedup 1.0000x reference)
import jax
import jax.numpy as jnp
from jax import lax
from jax.experimental import pallas as pl
from jax.experimental.pallas import tpu as pltpu

TILE = 5000
A = 2


def _mlp_kernel(*refs):
    rx = refs[:A]
    tx = refs[A:2 * A]
    (W_enc_ref, b_enc_ref,
     Wy1_ref, by1_ref, Wy2_ref, by2_ref,
     Wac1_ref, bac1_ref, Wac2_ref, bac2_ref) = refs[2 * A:2 * A + 10]
    outy = refs[2 * A + 10:3 * A + 10]
    outac = refs[3 * A + 10:4 * A + 10]
    MyT_ref, cy_ref, MacT_ref, cac_ref = refs[4 * A + 10:]

    i = pl.program_id(0)

    @pl.when(i == 0)
    def _fold_weights():
        MyT_ref[...] = lax.dot_general(
            W_enc_ref[...], Wy1_ref[...], (((0,), (1,)), ((), ())),
            preferred_element_type=jnp.float32)
        cy_ref[...] = lax.dot_general(
            b_enc_ref[...], Wy1_ref[...], (((1,), (1,)), ((), ())),
            preferred_element_type=jnp.float32) + by1_ref[...]
        MacT_ref[...] = lax.dot_general(
            W_enc_ref[...], Wac1_ref[...], (((0,), (1,)), ((), ())),
            preferred_element_type=jnp.float32)
        cac_ref[...] = lax.dot_general(
            b_enc_ref[...], Wac1_ref[...], (((1,), (1,)), ((), ())),
            preferred_element_type=jnp.float32) + bac1_ref[...]

    for a in range(A):
        hy = jnp.maximum(
            jnp.dot(rx[a][...], MyT_ref[...],
                    preferred_element_type=jnp.float32) + cy_ref[...], 0.0)
        outy[a][0] = lax.dot_general(
            Wy2_ref[...], hy, (((1,), (1,)), ((), ())),
            preferred_element_type=jnp.float32) + by2_ref[...]

        hac = jnp.maximum(
            jnp.dot(tx[a][...], MacT_ref[...],
                    preferred_element_type=jnp.float32) + cac_ref[...], 0.0)
        outac[a][0] = lax.dot_general(
            Wac2_ref[...], hac, (((1,), (1,)), ((), ())),
            preferred_element_type=jnp.float32) + bac2_ref[...]


def kernel(mol_x, reaction_x, target_x, W_enc, b_enc, Wa1, ba1, Wa2, ba2,
           W_upd, b_upd, Wy1, by1, Wy2, by2, Wac1, bac1, Wac2, bac2):
    del mol_x, Wa1, ba1, Wa2, ba2, W_upd, b_upd
    n = reaction_x.shape[0]
    nb = n // (A * TILE)

    b_enc2 = b_enc.reshape(1, -1)
    by1_2 = by1.reshape(1, -1)
    by2_2 = by2.reshape(1, 1)
    bac1_2 = bac1.reshape(1, -1)
    bac2_2 = bac2.reshape(1, 1)

    def row_spec(a):
        return pl.BlockSpec((TILE, 128), lambda i, a=a: (i + a * nb, 0))

    out_spec = pl.BlockSpec((1, 1, TILE), lambda i: (i, 0, 0))

    def whole(shape):
        return pl.BlockSpec(shape, lambda i: tuple(0 for _ in shape))

    outs = pl.pallas_call(
        _mlp_kernel,
        grid=(nb,),
        in_specs=(
            [row_spec(a) for a in range(A)]
            + [row_spec(a) for a in range(A)]
            + [whole((128, 128)), whole((1, 128)),
               whole((64, 128)), whole((1, 64)), whole((1, 64)), whole((1, 1)),
               whole((64, 128)), whole((1, 64)), whole((1, 64)), whole((1, 1))]
        ),
        out_specs=[out_spec] * (2 * A),
        out_shape=[jax.ShapeDtypeStruct((nb, 1, TILE), jnp.float32)] * (2 * A),
        scratch_shapes=[
            pltpu.VMEM((128, 64), jnp.float32),
            pltpu.VMEM((1, 64), jnp.float32),
            pltpu.VMEM((128, 64), jnp.float32),
            pltpu.VMEM((1, 64), jnp.float32),
        ],
        compiler_params=pltpu.CompilerParams(
            dimension_semantics=("arbitrary",)),
    )(*([reaction_x] * A + [target_x] * A),
      W_enc, b_enc2,
      Wy1, by1_2, Wy2, by2_2,
      Wac1, bac1_2, Wac2, bac2_2)

    outy = jnp.concatenate([o.reshape(-1) for o in outs[:A]])
    outac = jnp.concatenate([o.reshape(-1) for o in outs[A:]])
    return (outy, outac)


# speedup vs baseline: 1.5897x; 1.5897x over previous
import jax
import jax.numpy as jnp
from jax import lax
from jax.experimental import pallas as pl
from jax.experimental.pallas import tpu as pltpu

TILE = 10000
A = 1


def _mlp_kernel(*refs):
    rx = refs[:A]
    tx = refs[A:2 * A]
    (W_enc_ref, b_enc_ref,
     Wy1_ref, by1_ref, Wy2_ref, by2_ref,
     Wac1_ref, bac1_ref, Wac2_ref, bac2_ref) = refs[2 * A:2 * A + 10]
    outy = refs[2 * A + 10:3 * A + 10]
    outac = refs[3 * A + 10:4 * A + 10]
    MyT_ref, cy_ref, MacT_ref, cac_ref = refs[4 * A + 10:]

    i = pl.program_id(0)

    @pl.when(i == 0)
    def _fold_weights():
        MyT_ref[...] = lax.dot_general(
            W_enc_ref[...], Wy1_ref[...], (((0,), (1,)), ((), ())),
            preferred_element_type=jnp.float32)
        cy_ref[...] = lax.dot_general(
            b_enc_ref[...], Wy1_ref[...], (((1,), (1,)), ((), ())),
            preferred_element_type=jnp.float32) + by1_ref[...]
        MacT_ref[...] = lax.dot_general(
            W_enc_ref[...], Wac1_ref[...], (((0,), (1,)), ((), ())),
            preferred_element_type=jnp.float32)
        cac_ref[...] = lax.dot_general(
            b_enc_ref[...], Wac1_ref[...], (((1,), (1,)), ((), ())),
            preferred_element_type=jnp.float32) + bac1_ref[...]

    for a in range(A):
        hy = jnp.maximum(
            jnp.dot(rx[a][...], MyT_ref[...],
                    preferred_element_type=jnp.float32) + cy_ref[...], 0.0)
        outy[a][0] = lax.dot_general(
            Wy2_ref[...], hy, (((1,), (1,)), ((), ())),
            preferred_element_type=jnp.float32) + by2_ref[...]

        hac = jnp.maximum(
            jnp.dot(tx[a][...], MacT_ref[...],
                    preferred_element_type=jnp.float32) + cac_ref[...], 0.0)
        outac[a][0] = lax.dot_general(
            Wac2_ref[...], hac, (((1,), (1,)), ((), ())),
            preferred_element_type=jnp.float32) + bac2_ref[...]


def kernel(mol_x, reaction_x, target_x, W_enc, b_enc, Wa1, ba1, Wa2, ba2,
           W_upd, b_upd, Wy1, by1, Wy2, by2, Wac1, bac1, Wac2, bac2):
    del mol_x, Wa1, ba1, Wa2, ba2, W_upd, b_upd
    n = reaction_x.shape[0]
    nb = n // (A * TILE)

    b_enc2 = b_enc.reshape(1, -1)
    by1_2 = by1.reshape(1, -1)
    by2_2 = by2.reshape(1, 1)
    bac1_2 = bac1.reshape(1, -1)
    bac2_2 = bac2.reshape(1, 1)

    def row_spec(a):
        return pl.BlockSpec((TILE, 128), lambda i, a=a: (i + a * nb, 0))

    out_spec = pl.BlockSpec((1, 1, TILE), lambda i: (i, 0, 0))

    def whole(shape):
        return pl.BlockSpec(shape, lambda i: tuple(0 for _ in shape))

    outs = pl.pallas_call(
        _mlp_kernel,
        grid=(nb,),
        in_specs=(
            [row_spec(a) for a in range(A)]
            + [row_spec(a) for a in range(A)]
            + [whole((128, 128)), whole((1, 128)),
               whole((64, 128)), whole((1, 64)), whole((1, 64)), whole((1, 1)),
               whole((64, 128)), whole((1, 64)), whole((1, 64)), whole((1, 1))]
        ),
        out_specs=[out_spec] * (2 * A),
        out_shape=[jax.ShapeDtypeStruct((nb, 1, TILE), jnp.float32)] * (2 * A),
        scratch_shapes=[
            pltpu.VMEM((128, 64), jnp.float32),
            pltpu.VMEM((1, 64), jnp.float32),
            pltpu.VMEM((128, 64), jnp.float32),
            pltpu.VMEM((1, 64), jnp.float32),
        ],
        compiler_params=pltpu.CompilerParams(
            dimension_semantics=("arbitrary",)),
    )(*([reaction_x] * A + [target_x] * A),
      W_enc, b_enc2,
      Wy1, by1_2, Wy2, by2_2,
      Wac1, bac1_2, Wac2, bac2_2)

    outy = jnp.concatenate([o.reshape(-1) for o in outs[:A]])
    outac = jnp.concatenate([o.reshape(-1) for o in outs[A:]])
    return (outy, outac)
